# baseline (speedup 1.0000x reference)
import jax
import jax.numpy as jnp
from jax.experimental import pallas as pl
from jax.experimental.pallas import tpu as pltpu

_GK = 2048


def _mm_body(x_ref, o_ref):
    br, bh = x_ref.shape
    gk = min(_GK, bh)
    lane = jax.lax.broadcasted_iota(jnp.int32, (br, gk), 1)
    quad = (lane & 3) == 0
    lane128 = jax.lax.broadcasted_iota(jnp.int32, (br, 128), 1)
    pat = (lane128 + lane128) & 127
    half = lane128 < 64
    for g in range(bh // gk):
        x = x_ref[:, g * gk:(g + 1) * gk]
        x1 = pltpu.roll(x, gk - 1, 1)
        x2 = pltpu.roll(x, gk - 2, 1)
        x3 = pltpu.roll(x, gk - 3, 1)
        mx = jnp.maximum(jnp.maximum(x, x1), jnp.maximum(x2, x3))
        mn = jnp.minimum(jnp.minimum(x, x1), jnp.minimum(x2, x3))
        idxf = jnp.where(x >= mx, 0.0,
                         jnp.where(x1 >= mx, 1.0,
                                   jnp.where(x2 >= mx, 2.0, 3.0)))
        mn2 = pltpu.roll(mn, 2, 1)
        t = jnp.where(quad, idxf, mn2)
        chunks = []
        for s in range(gk // 256):
            g0 = jnp.take_along_axis(t[:, 256 * s:256 * s + 128], pat, axis=1)
            g1 = jnp.take_along_axis(t[:, 256 * s + 128:256 * s + 256], pat, axis=1)
            chunks.append(jnp.where(half, g0, g1))
        o_ref[:, g * gk // 2:(g + 1) * gk // 2] = jnp.concatenate(chunks, axis=1)


def kernel(feature_map):
    C, H = feature_map.shape
    BR = 8
    return pl.pallas_call(
        _mm_body,
        grid=(C // BR,),
        in_specs=[pl.BlockSpec((BR, H), lambda i: (i, 0))],
        out_specs=pl.BlockSpec((BR, H // 2), lambda i: (i, 0)),
        out_shape=jax.ShapeDtypeStruct((C, H // 2), feature_map.dtype),
        compiler_params=pltpu.CompilerParams(
            dimension_semantics=("parallel",),
        ),
    )(feature_map)


# speedup vs baseline: 1.6555x; 1.6555x over previous
import jax
import jax.numpy as jnp
from jax.experimental import pallas as pl
from jax.experimental.pallas import tpu as pltpu


def _half_t(x):
    x1 = pltpu.roll(x, 127, 1)
    x2 = pltpu.roll(x, 126, 1)
    x3 = pltpu.roll(x, 125, 1)
    mx = jnp.maximum(jnp.maximum(x, x1), jnp.maximum(x2, x3))
    mn = jnp.minimum(jnp.minimum(x, x1), jnp.minimum(x2, x3))
    idxf = jnp.where(x >= mx, 0.0,
                     jnp.where(x1 >= mx, 1.0,
                               jnp.where(x2 >= mx, 2.0, 3.0)))
    mn2 = pltpu.roll(mn, 2, 1)
    lanes = jax.lax.broadcasted_iota(jnp.int32, x.shape, 1)
    return jnp.where((lanes & 3) == 0, idxf, mn2)


def _mm_body(x_ref, o_ref):
    br = x_ref.shape[0]
    t0 = _half_t(x_ref[:, 0:128])
    t1 = _half_t(x_ref[:, 128:256])
    lane128 = jax.lax.broadcasted_iota(jnp.int32, (br, 128), 1)
    pat = (lane128 + lane128) & 127
    g0 = jnp.take_along_axis(t0, pat, axis=1)
    g1 = jnp.take_along_axis(t1, pat, axis=1)
    o_ref[...] = jnp.where(lane128 < 64, g0, g1)


def kernel(feature_map):
    C, H = feature_map.shape
    BR = 512
    BH = 256
    return pl.pallas_call(
        _mm_body,
        grid=(C // BR, H // BH),
        in_specs=[pl.BlockSpec((BR, BH), lambda i, j: (i, j))],
        out_specs=pl.BlockSpec((BR, BH // 2), lambda i, j: (i, j)),
        out_shape=jax.ShapeDtypeStruct((C, H // 2), feature_map.dtype),
        compiler_params=pltpu.CompilerParams(
            dimension_semantics=("parallel", "arbitrary"),
        ),
    )(feature_map)


# speedup vs baseline: 2.7999x; 1.6912x over previous
import jax
import jax.numpy as jnp
from jax.experimental import pallas as pl
from jax.experimental.pallas import tpu as pltpu


def _half_am(x):
    x1 = pltpu.roll(x, 127, 1)
    x2 = pltpu.roll(x, 126, 1)
    x3 = pltpu.roll(x, 125, 1)
    mx = jnp.maximum(jnp.maximum(x, x1), jnp.maximum(x2, x3))
    mn = jnp.minimum(jnp.minimum(x, x1), jnp.minimum(x2, x3))
    a = jnp.where(x >= mx, 0.0,
                  jnp.where(x1 >= mx, 1.0,
                            jnp.where(x2 >= mx, 2.0, 3.0)))
    return a.astype(jnp.bfloat16), mn


def _mm_body(x_ref, o_ref):
    br = x_ref.shape[0]
    a0, m0 = _half_am(x_ref[:, 0:128])
    a1, m1 = _half_am(x_ref[:, 128:256])

    j = jax.lax.broadcasted_iota(jnp.int32, (128, 128), 0)
    l = jax.lax.broadcasted_iota(jnp.int32, (128, 128), 1)
    ev = (l & 1) == 0
    pa0 = jnp.where((l < 64) & ev & (j == 2 * l), 1.0, 0.0).astype(jnp.bfloat16)
    pa1 = jnp.where((l >= 64) & ev & (j == 2 * (l - 64)), 1.0, 0.0).astype(jnp.bfloat16)
    outa = (jnp.dot(a0, pa0, preferred_element_type=jnp.float32)
            + jnp.dot(a1, pa1, preferred_element_type=jnp.float32))

    lane128 = jax.lax.broadcasted_iota(jnp.int32, (br, 128), 1)
    patm = (lane128 + lane128 - 2) & 127
    gm0 = jnp.take_along_axis(m0, patm, axis=1)
    gm1 = jnp.take_along_axis(m1, patm, axis=1)
    gm = jnp.where(lane128 < 64, gm0, gm1)

    o_ref[...] = jnp.where((lane128 & 1) == 0, outa, gm)


def kernel(feature_map):
    C, H = feature_map.shape
    BR = 512
    BH = 256
    return pl.pallas_call(
        _mm_body,
        grid=(C // BR, H // BH),
        in_specs=[pl.BlockSpec((BR, BH), lambda i, j: (i, j))],
        out_specs=pl.BlockSpec((BR, BH // 2), lambda i, j: (i, j)),
        out_shape=jax.ShapeDtypeStruct((C, H // 2), feature_map.dtype),
        compiler_params=pltpu.CompilerParams(
            dimension_semantics=("parallel", "arbitrary"),
        ),
    )(feature_map)
